# baseline (speedup 1.0000x reference)
import functools

import jax
import jax.numpy as jnp
from jax import lax
from jax.experimental import pallas as pl
from jax.experimental.pallas import tpu as pltpu
from jax.experimental.pallas import tpu_sc as plsc

DIM = 64
LANES = 16
IDX_CHUNK = 128


@functools.lru_cache(maxsize=None)
def _make_sc_kernel(B):
    info = plsc.get_sparse_core_info()
    NC, NS = info.num_cores, info.num_subcores
    NW = NC * NS
    bw = B // NW
    n_chunk = bw // IDX_CHUNK
    mesh = plsc.VectorSubcoreMesh(core_axis_name="c", subcore_axis_name="s")

    @functools.partial(
        pl.kernel,
        mesh=mesh,
        compiler_params=pltpu.CompilerParams(
            needs_layout_passes=False, use_tc_tiling_on_sc=False),
        out_type=(
            jax.ShapeDtypeStruct((B,), jnp.float32),
            jax.ShapeDtypeStruct((B,), jnp.float32),
        ),
        scratch_types=[
            pltpu.VMEM((n_chunk, IDX_CHUNK), jnp.int32),
            pltpu.VMEM((n_chunk, IDX_CHUNK), jnp.int32),
            pltpu.VMEM((n_chunk, IDX_CHUNK), jnp.int32),
            pltpu.VMEM((bw, DIM), jnp.float32),
            pltpu.VMEM((bw, DIM), jnp.float32),
            pltpu.VMEM((bw, DIM), jnp.float32),
            pltpu.VMEM((bw,), jnp.float32),
            pltpu.VMEM((bw,), jnp.float32),
            pltpu.SemaphoreType.DMA,
        ],
    )
    def sc_kernel(u_hbm, i_hbm, n_hbm, ue_hbm, ie_hbm, pos_hbm, neg_hbm,
                  uidx, iidx, nidx, urows, irows, nrows, opos, oneg, sem):
        wid = lax.axis_index("s") * NC + lax.axis_index("c")
        base = wid * bw

        idx_cps = []
        for j in range(n_chunk):
            src = pl.ds(base + j * IDX_CHUNK, IDX_CHUNK)
            idx_cps.append(pltpu.async_copy(u_hbm.at[src], uidx.at[j], sem))
            idx_cps.append(pltpu.async_copy(i_hbm.at[src], iidx.at[j], sem))
            idx_cps.append(pltpu.async_copy(n_hbm.at[src], nidx.at[j], sem))
        for c in idx_cps:
            c.wait()

        row_cps = []
        for j in range(n_chunk):
            dst = pl.ds(j * IDX_CHUNK, IDX_CHUNK)
            row_cps.append(pltpu.async_copy(ue_hbm.at[uidx.at[j]], urows.at[dst], sem))
            row_cps.append(pltpu.async_copy(ie_hbm.at[iidx.at[j]], irows.at[dst], sem))
            row_cps.append(pltpu.async_copy(ie_hbm.at[nidx.at[j]], nrows.at[dst], sem))
        for c in row_cps:
            c.wait()

        iota = lax.iota(jnp.int32, LANES)
        n_grp = bw // LANES

        def body(g, carry):
            row = iota + g * LANES
            ap = [jnp.zeros((LANES,), jnp.float32) for _ in range(4)]
            an = [jnp.zeros((LANES,), jnp.float32) for _ in range(4)]
            for d in range(DIM):
                col = jnp.full((LANES,), d, jnp.int32)
                uv = plsc.load_gather(urows, [row, col])
                iv = plsc.load_gather(irows, [row, col])
                nv = plsc.load_gather(nrows, [row, col])
                ap[d % 4] = ap[d % 4] + uv * iv
                an[d % 4] = an[d % 4] + uv * nv
            opos[pl.ds(g * LANES, LANES)] = (ap[0] + ap[1]) + (ap[2] + ap[3])
            oneg[pl.ds(g * LANES, LANES)] = (an[0] + an[1]) + (an[2] + an[3])
            return carry

        lax.fori_loop(0, n_grp, body, 0)

        pltpu.sync_copy(opos, pos_hbm.at[pl.ds(base, bw)])
        pltpu.sync_copy(oneg, neg_hbm.at[pl.ds(base, bw)])

    return sc_kernel


def kernel(u, i, neg_i, user_emb, item_emb):
    B = u.shape[0]
    sc = _make_sc_kernel(B)
    return sc(u.astype(jnp.int32), i.astype(jnp.int32), neg_i.astype(jnp.int32),
              user_emb, item_emb)


# speedup vs baseline: 1.0038x; 1.0038x over previous
import functools

import jax
import jax.numpy as jnp
from jax import lax
from jax.experimental import pallas as pl
from jax.experimental.pallas import tpu as pltpu
from jax.experimental.pallas import tpu_sc as plsc

DIM = 64
LANES = 16
CHUNK = 128


@functools.lru_cache(maxsize=None)
def _make_sc_kernel(B):
    info = plsc.get_sparse_core_info()
    NC, NS = info.num_cores, info.num_subcores
    NW = NC * NS
    bw = B // NW
    n_chunk = bw // CHUNK
    n_grp = CHUNK // LANES
    mesh = plsc.VectorSubcoreMesh(core_axis_name="c", subcore_axis_name="s")

    @functools.partial(
        pl.kernel,
        mesh=mesh,
        compiler_params=pltpu.CompilerParams(needs_layout_passes=False),
        out_type=(
            jax.ShapeDtypeStruct((B,), jnp.float32),
            jax.ShapeDtypeStruct((B,), jnp.float32),
        ),
        scratch_types=[
            pltpu.VMEM((n_chunk, CHUNK), jnp.int32),
            pltpu.VMEM((n_chunk, CHUNK), jnp.int32),
            pltpu.VMEM((n_chunk, CHUNK), jnp.int32),
            pltpu.VMEM((n_chunk, CHUNK), jnp.int32),
            pltpu.VMEM((n_chunk, CHUNK), jnp.int32),
            pltpu.VMEM((n_chunk, CHUNK), jnp.int32),
            pltpu.VMEM((n_chunk, CHUNK), jnp.int32),
            pltpu.VMEM((n_chunk, CHUNK), jnp.int32),
            pltpu.VMEM((n_chunk, CHUNK), jnp.int32),
            pltpu.VMEM((2 * CHUNK, 2 * DIM), jnp.float32),
            pltpu.VMEM((2 * CHUNK, 2 * DIM), jnp.float32),
            pltpu.VMEM((2 * CHUNK, 2 * DIM), jnp.float32),
            pltpu.VMEM((bw,), jnp.float32),
            pltpu.VMEM((bw,), jnp.float32),
            pltpu.SemaphoreType.DMA,
            pltpu.SemaphoreType.DMA,
            pltpu.SemaphoreType.DMA,
        ],
    )
    def sc_kernel(u_hbm, i_hbm, n_hbm, ue_hbm, ie_hbm, pos_hbm, neg_hbm,
                  uidx, iidx, nidx, urow, irow, nrow, ucol, icol, ncol,
                  ubuf, ibuf, nbuf, opos, oneg, semi, sem0, sem1):
        wid = lax.axis_index("s") * NC + lax.axis_index("c")
        base = wid * bw

        idx_cps = []
        for j in range(n_chunk):
            src = pl.ds(base + j * CHUNK, CHUNK)
            idx_cps.append(pltpu.async_copy(u_hbm.at[src], uidx.at[j], semi))
            idx_cps.append(pltpu.async_copy(i_hbm.at[src], iidx.at[j], semi))
            idx_cps.append(pltpu.async_copy(n_hbm.at[src], nidx.at[j], semi))
        for c in idx_cps:
            c.wait()

        for src_ref, row_ref, col_ref in ((uidx, urow, ucol),
                                          (iidx, irow, icol),
                                          (nidx, nrow, ncol)):
            def split(s, carry, src_ref=src_ref, row_ref=row_ref,
                      col_ref=col_ref):
                j = s // (CHUNK // LANES)
                o = (s % (CHUNK // LANES)) * LANES
                v = src_ref[j, pl.ds(o, LANES)]
                row_ref[j, pl.ds(o, LANES)] = lax.shift_right_logical(v, 1)
                col_ref[j, pl.ds(o, LANES)] = lax.shift_left(
                    lax.bitwise_and(v, 1), 6)
                return carry
            lax.fori_loop(0, n_chunk * (CHUNK // LANES), split, 0)

        sems = (sem0, sem1)

        def fire(c):
            slot = c % 2
            dst = pl.ds(slot * CHUNK, CHUNK)
            sem = sems[slot]
            return [
                pltpu.async_copy(ue_hbm.at[urow.at[c]], ubuf.at[dst], sem),
                pltpu.async_copy(ie_hbm.at[irow.at[c]], ibuf.at[dst], sem),
                pltpu.async_copy(ie_hbm.at[nrow.at[c]], nbuf.at[dst], sem),
            ]

        iota = lax.iota(jnp.int32, LANES)
        inflight = fire(0)
        for c in range(n_chunk):
            for cp in inflight:
                cp.wait()
            if c + 1 < n_chunk:
                inflight = fire(c + 1)
            slot = c % 2

            def body(g, carry, c=c, slot=slot):
                brow = iota + (slot * CHUNK + g * LANES)
                ucolv = ucol[c, pl.ds(g * LANES, LANES)]
                icolv = icol[c, pl.ds(g * LANES, LANES)]
                ncolv = ncol[c, pl.ds(g * LANES, LANES)]
                ap0 = jnp.zeros((LANES,), jnp.float32)
                ap1 = jnp.zeros((LANES,), jnp.float32)
                an0 = jnp.zeros((LANES,), jnp.float32)
                an1 = jnp.zeros((LANES,), jnp.float32)
                for d in range(DIM):
                    uv = plsc.load_gather(ubuf, [brow, ucolv + d])
                    iv = plsc.load_gather(ibuf, [brow, icolv + d])
                    nv = plsc.load_gather(nbuf, [brow, ncolv + d])
                    if d % 2 == 0:
                        ap0 = ap0 + uv * iv
                        an0 = an0 + uv * nv
                    else:
                        ap1 = ap1 + uv * iv
                        an1 = an1 + uv * nv
                out = pl.ds((c * n_grp + g) * LANES, LANES)
                opos[out] = ap0 + ap1
                oneg[out] = an0 + an1
                return carry

            lax.fori_loop(0, n_grp, body, 0)

        pltpu.sync_copy(opos, pos_hbm.at[pl.ds(base, bw)])
        pltpu.sync_copy(oneg, neg_hbm.at[pl.ds(base, bw)])

    return sc_kernel


def kernel(u, i, neg_i, user_emb, item_emb):
    B = u.shape[0]
    sc = _make_sc_kernel(B)
    ue2 = user_emb.reshape(-1, 2 * DIM)
    ie2 = item_emb.reshape(-1, 2 * DIM)
    return sc(u.astype(jnp.int32), i.astype(jnp.int32),
              neg_i.astype(jnp.int32), ue2, ie2)
